# initial kernel scaffold (unmeasured)
import jax
import jax.numpy as jnp
from jax import lax
from jax.experimental import pallas as pl
from jax.experimental.pallas import tpu as pltpu

NY = 4
CH = 1024


def kernel(partial, resid, gamma):
    _, M, D = partial.shape
    B = M // NY
    n_ch = B // CH
    gamma2d = gamma.reshape(1, D)

    def body(partial_ref, resid_ref, gamma_ref, out_ref,
             recv_hbm, send_hbm, va, vb, vc,
             copy_sems, rs_send, rs_recv, ag_send, ag_recv):
        my_x = lax.axis_index("x")
        my_y = lax.axis_index("y")
        my_z = lax.axis_index("z")
        right = (my_x, (my_y + 1) % NY, my_z)
        left = (my_x, (my_y + NY - 1) % NY, my_z)

        barrier = pltpu.get_barrier_semaphore()
        for nbr in (left, right):
            pl.semaphore_signal(barrier, inc=1, device_id=nbr,
                                device_id_type=pl.DeviceIdType.MESH)
        pl.semaphore_wait(barrier, 2)

        b0 = (my_y + NY - 1) % NY
        rdma = pltpu.make_async_remote_copy(
            src_ref=partial_ref.at[0, pl.ds(b0 * B, B), :],
            dst_ref=recv_hbm.at[0],
            send_sem=rs_send.at[0],
            recv_sem=rs_recv.at[0],
            device_id=right,
            device_id_type=pl.DeviceIdType.MESH,
        )
        rdma.start()
        rdma.wait()

        for s in (1, 2):
            b = (my_y + NY - 1 - s) % NY
            for c in range(n_ch):
                r0 = c * CH
                cp_a = pltpu.make_async_copy(
                    recv_hbm.at[s - 1, pl.ds(r0, CH), :], va, copy_sems.at[0])
                cp_b = pltpu.make_async_copy(
                    partial_ref.at[0, pl.ds(b * B + r0, CH), :], vb,
                    copy_sems.at[1])
                cp_a.start()
                cp_b.start()
                cp_a.wait()
                cp_b.wait()
                vc[...] = va[...] + vb[...]
                cp_c = pltpu.make_async_copy(
                    vc, send_hbm.at[s - 1, pl.ds(r0, CH), :], copy_sems.at[2])
                cp_c.start()
                cp_c.wait()
            rdma = pltpu.make_async_remote_copy(
                src_ref=send_hbm.at[s - 1],
                dst_ref=recv_hbm.at[s],
                send_sem=rs_send.at[s],
                recv_sem=rs_recv.at[s],
                device_id=right,
                device_id_type=pl.DeviceIdType.MESH,
            )
            rdma.start()
            rdma.wait()

        for c in range(n_ch):
            r0 = my_y * B + c * CH
            cp_a = pltpu.make_async_copy(
                recv_hbm.at[2, pl.ds(c * CH, CH), :], va, copy_sems.at[0])
            cp_b = pltpu.make_async_copy(
                partial_ref.at[0, pl.ds(r0, CH), :], vb, copy_sems.at[1])
            cp_c = pltpu.make_async_copy(
                resid_ref.at[pl.ds(r0, CH), :], vc, copy_sems.at[2])
            cp_a.start()
            cp_b.start()
            cp_c.start()
            cp_a.wait()
            cp_b.wait()
            cp_c.wait()
            y = va[...] + vb[...] + vc[...]
            rms = jnp.sqrt(jnp.mean(y * y, axis=-1, keepdims=True) + 1e-6)
            va[...] = y / rms * gamma_ref[...]
            cp_o = pltpu.make_async_copy(
                va, out_ref.at[pl.ds(r0, CH), :], copy_sems.at[0])
            cp_o.start()
            cp_o.wait()

        for t in range(NY - 1):
            g = (my_y + NY - t) % NY
            rdma = pltpu.make_async_remote_copy(
                src_ref=out_ref.at[pl.ds(g * B, B), :],
                dst_ref=out_ref.at[pl.ds(g * B, B), :],
                send_sem=ag_send.at[t],
                recv_sem=ag_recv.at[t],
                device_id=right,
                device_id_type=pl.DeviceIdType.MESH,
            )
            rdma.start()
            rdma.wait()

    return pl.pallas_call(
        body,
        out_shape=jax.ShapeDtypeStruct((M, D), jnp.float32),
        in_specs=[
            pl.BlockSpec(memory_space=pltpu.MemorySpace.ANY),
            pl.BlockSpec(memory_space=pltpu.MemorySpace.ANY),
            pl.BlockSpec(memory_space=pltpu.MemorySpace.VMEM),
        ],
        out_specs=pl.BlockSpec(memory_space=pltpu.MemorySpace.ANY),
        scratch_shapes=[
            pltpu.MemorySpace.ANY((3, B, D), jnp.float32),
            pltpu.MemorySpace.ANY((2, B, D), jnp.float32),
            pltpu.VMEM((CH, D), jnp.float32),
            pltpu.VMEM((CH, D), jnp.float32),
            pltpu.VMEM((CH, D), jnp.float32),
            pltpu.SemaphoreType.DMA((3,)),
            pltpu.SemaphoreType.DMA((3,)),
            pltpu.SemaphoreType.DMA((3,)),
            pltpu.SemaphoreType.DMA((3,)),
            pltpu.SemaphoreType.DMA((3,)),
        ],
        compiler_params=pltpu.CompilerParams(collective_id=0),
    )(partial, resid, gamma2d)


# baseline (device time: 1209236 ns/iter reference)
import jax
import jax.numpy as jnp
from jax import lax
from jax.experimental import pallas as pl
from jax.experimental.pallas import tpu as pltpu

NY = 4
CH = 1024


def kernel(partial, resid, gamma):
    _, M, D = partial.shape
    B = M // NY
    n_ch = B // CH
    gamma2d = gamma.reshape(1, D)

    def body(partial_ref, resid_ref, gamma_ref, out_ref, recv_hbm, send_hbm,
             va, vb, vc,
             copy_sems, rs_send, rs_recv, ag_send, ag_recv):
        my_x = lax.axis_index("x")
        my_y = lax.axis_index("y")
        my_z = lax.axis_index("z")
        right = (my_x, (my_y + 1) % NY, my_z)
        left = (my_x, (my_y + NY - 1) % NY, my_z)

        barrier = pltpu.get_barrier_semaphore()
        for nbr in (left, right):
            pl.semaphore_signal(barrier, inc=1, device_id=nbr,
                                device_id_type=pl.DeviceIdType.MESH)
        pl.semaphore_wait(barrier, 2)

        b0 = (my_y + NY - 1) % NY
        rdma = pltpu.make_async_remote_copy(
            src_ref=partial_ref.at[0, pl.ds(b0 * B, B), :],
            dst_ref=recv_hbm.at[0],
            send_sem=rs_send.at[0],
            recv_sem=rs_recv.at[0],
            device_id=right,
            device_id_type=pl.DeviceIdType.MESH,
        )
        rdma.start()
        rdma.wait()

        for s in (1, 2):
            b = (my_y + NY - 1 - s) % NY
            for c in range(n_ch):
                r0 = c * CH
                cp_a = pltpu.make_async_copy(
                    recv_hbm.at[s - 1, pl.ds(r0, CH), :], va, copy_sems.at[0])
                cp_b = pltpu.make_async_copy(
                    partial_ref.at[0, pl.ds(b * B + r0, CH), :], vb,
                    copy_sems.at[1])
                cp_a.start()
                cp_b.start()
                cp_a.wait()
                cp_b.wait()
                vc[...] = va[...] + vb[...]
                cp_c = pltpu.make_async_copy(
                    vc, send_hbm.at[s - 1, pl.ds(r0, CH), :], copy_sems.at[2])
                cp_c.start()
                cp_c.wait()
            rdma = pltpu.make_async_remote_copy(
                src_ref=send_hbm.at[s - 1],
                dst_ref=recv_hbm.at[s],
                send_sem=rs_send.at[s],
                recv_sem=rs_recv.at[s],
                device_id=right,
                device_id_type=pl.DeviceIdType.MESH,
            )
            rdma.start()
            rdma.wait()

        for c in range(n_ch):
            r0 = my_y * B + c * CH
            cp_a = pltpu.make_async_copy(
                recv_hbm.at[2, pl.ds(c * CH, CH), :], va, copy_sems.at[0])
            cp_b = pltpu.make_async_copy(
                partial_ref.at[0, pl.ds(r0, CH), :], vb, copy_sems.at[1])
            cp_c = pltpu.make_async_copy(
                resid_ref.at[pl.ds(r0, CH), :], vc, copy_sems.at[2])
            cp_a.start()
            cp_b.start()
            cp_c.start()
            cp_a.wait()
            cp_b.wait()
            cp_c.wait()
            y = va[...] + vb[...] + vc[...]
            rms = jnp.sqrt(jnp.mean(y * y, axis=-1, keepdims=True) + 1e-6)
            va[...] = y / rms * gamma_ref[...]
            cp_o = pltpu.make_async_copy(
                va, out_ref.at[pl.ds(r0, CH), :], copy_sems.at[0])
            cp_o.start()
            cp_o.wait()

        for t in range(NY - 1):
            g = (my_y + NY - t) % NY
            rdma = pltpu.make_async_remote_copy(
                src_ref=out_ref.at[pl.ds(g * B, B), :],
                dst_ref=out_ref.at[pl.ds(g * B, B), :],
                send_sem=ag_send.at[t],
                recv_sem=ag_recv.at[t],
                device_id=right,
                device_id_type=pl.DeviceIdType.MESH,
            )
            rdma.start()
            rdma.wait()

    out, _, _ = pl.pallas_call(
        body,
        out_shape=[
            jax.ShapeDtypeStruct((M, D), jnp.float32),
            jax.ShapeDtypeStruct((3, B, D), jnp.float32),
            jax.ShapeDtypeStruct((2, B, D), jnp.float32),
        ],
        in_specs=[
            pl.BlockSpec(memory_space=pl.ANY),
            pl.BlockSpec(memory_space=pl.ANY),
            pl.BlockSpec(memory_space=pltpu.MemorySpace.VMEM),
        ],
        out_specs=[
            pl.BlockSpec(memory_space=pl.ANY),
            pl.BlockSpec(memory_space=pl.ANY),
            pl.BlockSpec(memory_space=pl.ANY),
        ],
        scratch_shapes=[
            pltpu.VMEM((CH, D), jnp.float32),
            pltpu.VMEM((CH, D), jnp.float32),
            pltpu.VMEM((CH, D), jnp.float32),
            pltpu.SemaphoreType.DMA((3,)),
            pltpu.SemaphoreType.DMA((3,)),
            pltpu.SemaphoreType.DMA((3,)),
            pltpu.SemaphoreType.DMA((3,)),
            pltpu.SemaphoreType.DMA((3,)),
        ],
        compiler_params=pltpu.CompilerParams(
            collective_id=0, vmem_limit_bytes=60 * 1024 * 1024),
    )(partial, resid, gamma2d)
    return out
